# baseline (speedup 1.0000x reference)
import functools

import jax
import jax.numpy as jnp
from jax import lax
from jax.experimental import pallas as pl
from jax.experimental.pallas import tpu as pltpu
from jax.experimental.pallas import tpu_sc as plsc

N, L, D = 32, 1024, 256
B = N * L
LANES = 16
NUM_WORKERS = 32
B_PER_W = B // NUM_WORKERS
CHUNK = 128
N_CHUNKS = B_PER_W // CHUNK


def _make_lookup():
  mesh = plsc.VectorSubcoreMesh(core_axis_name="c", subcore_axis_name="s")

  @functools.partial(
      pl.kernel,
      mesh=mesh,
      out_type=jax.ShapeDtypeStruct((B, D), jnp.float32),
      scratch_types=[
          pltpu.VMEM((B_PER_W,), jnp.int32),
          pltpu.VMEM((B_PER_W,), jnp.int32),
          pltpu.VMEM((CHUNK, D), jnp.float32),
          pltpu.SemaphoreType.DMA,
      ],
  )
  def lookup(aa_hbm, mask_hbm, table_hbm, out_hbm, idx_v, mask_v, rows_v, sem):
    wid = lax.axis_index("s") * 2 + lax.axis_index("c")
    base = wid * B_PER_W
    pltpu.sync_copy(aa_hbm.at[pl.ds(base, B_PER_W)], idx_v)
    pltpu.sync_copy(mask_hbm.at[pl.ds(base, B_PER_W)], mask_v)
    for j in range(B_PER_W // LANES):
      sl = pl.ds(j * LANES, LANES)
      a = idx_v[sl]
      m = mask_v[sl]
      idx_v[sl] = jnp.where(m == 0, jnp.int32(7), a)
    for c in range(N_CHUNKS):
      pltpu.async_copy(
          table_hbm.at[idx_v.at[pl.ds(c * CHUNK, CHUNK)]], rows_v, sem
      ).wait()
      pltpu.sync_copy(rows_v, out_hbm.at[pl.ds(base + c * CHUNK, CHUNK)])

  return lookup


_lookup = _make_lookup()


def kernel(aa, res_nb, chain_nb, pos_atoms, mask_atoms, fragment_type, emb_table):
  aa_flat = aa.reshape(B).astype(jnp.int32)
  mask_flat = mask_atoms[:, :, 0].reshape(B).astype(jnp.int32)
  table_ext = jnp.concatenate(
      [emb_table.astype(jnp.float32), jnp.zeros((1, D), jnp.float32)], axis=0
  )
  out = _lookup(aa_flat, mask_flat, table_ext)
  return out.reshape(N, L, D)


# speedup vs baseline: 1.0005x; 1.0005x over previous
import functools

import jax
import jax.numpy as jnp
from jax import lax
from jax.experimental import pallas as pl
from jax.experimental.pallas import tpu as pltpu
from jax.experimental.pallas import tpu_sc as plsc

N, L, D = 32, 1024, 256
B = N * L
LANES = 16
NUM_WORKERS = 32
B_PER_W = B // NUM_WORKERS
CHUNK = 128
N_CHUNKS = B_PER_W // CHUNK


def _make_lookup():
  mesh = plsc.VectorSubcoreMesh(core_axis_name="c", subcore_axis_name="s")

  @functools.partial(
      pl.kernel,
      mesh=mesh,
      out_type=jax.ShapeDtypeStruct((B, D), jnp.float32),
      scratch_types=[
          pltpu.VMEM((B_PER_W,), jnp.int32),
          pltpu.VMEM((B_PER_W,), jnp.int32),
          pltpu.VMEM((CHUNK, D), jnp.float32),
          pltpu.VMEM((CHUNK, D), jnp.float32),
          pltpu.SemaphoreType.DMA,
          pltpu.SemaphoreType.DMA,
          pltpu.SemaphoreType.DMA,
          pltpu.SemaphoreType.DMA,
      ],
  )
  def lookup(aa_hbm, mask_hbm, table_hbm, out_hbm, idx_v, mask_v, rows_a,
             rows_b, sg_a, sg_b, ss_a, ss_b):
    wid = lax.axis_index("s") * 2 + lax.axis_index("c")
    base = wid * B_PER_W
    pltpu.sync_copy(aa_hbm.at[pl.ds(base, B_PER_W)], idx_v)
    pltpu.sync_copy(mask_hbm.at[pl.ds(base, B_PER_W)], mask_v)
    for j in range(B_PER_W // LANES):
      sl = pl.ds(j * LANES, LANES)
      a = idx_v[sl]
      m = mask_v[sl]
      idx_v[sl] = jnp.where(m == 0, jnp.int32(7), a)
    rows = (rows_a, rows_b)
    sg = (sg_a, sg_b)
    ss = (ss_a, ss_b)
    g = [None] * N_CHUNKS
    s = [None] * N_CHUNKS
    for c in range(N_CHUNKS):
      b = c % 2
      if c >= 2:
        s[c - 2].wait()
      g[c] = pltpu.async_copy(
          table_hbm.at[idx_v.at[pl.ds(c * CHUNK, CHUNK)]], rows[b], sg[b]
      )
      if c >= 1:
        p = c - 1
        g[p].wait()
        s[p] = pltpu.async_copy(
            rows[p % 2], out_hbm.at[pl.ds(base + p * CHUNK, CHUNK)], ss[p % 2]
        )
    last = N_CHUNKS - 1
    g[last].wait()
    s[last] = pltpu.async_copy(
        rows[last % 2], out_hbm.at[pl.ds(base + last * CHUNK, CHUNK)],
        ss[last % 2]
    )
    s[last - 1].wait()
    s[last].wait()

  return lookup


_lookup = _make_lookup()


def kernel(aa, res_nb, chain_nb, pos_atoms, mask_atoms, fragment_type, emb_table):
  aa_flat = aa.reshape(B).astype(jnp.int32)
  mask_flat = mask_atoms[:, :, 0].reshape(B).astype(jnp.int32)
  table_ext = jnp.concatenate(
      [emb_table.astype(jnp.float32), jnp.zeros((1, D), jnp.float32)], axis=0
  )
  out = _lookup(aa_flat, mask_flat, table_ext)
  return out.reshape(N, L, D)


# speedup vs baseline: 3.4514x; 3.4497x over previous
import functools

import jax
import jax.numpy as jnp
from jax import lax
from jax.experimental import pallas as pl
from jax.experimental.pallas import tpu as pltpu
from jax.experimental.pallas import tpu_sc as plsc

N, L, D = 32, 1024, 256
B = N * L
LANES = 16
NUM_WORKERS = 32
B_PER_W = B // NUM_WORKERS
CHUNK = 128
N_CHUNKS = B_PER_W // CHUNK


def _make_lookup():
  mesh = plsc.VectorSubcoreMesh(core_axis_name="c", subcore_axis_name="s")

  @functools.partial(
      pl.kernel,
      mesh=mesh,
      out_type=jax.ShapeDtypeStruct((B, D), jnp.float32),
      scratch_types=[
          pltpu.VMEM((B_PER_W,), jnp.int32),
          pltpu.VMEM((B_PER_W,), jnp.int32),
          pltpu.VMEM((CHUNK, D), jnp.float32),
          pltpu.VMEM((CHUNK, D), jnp.float32),
          pltpu.SemaphoreType.DMA,
          pltpu.SemaphoreType.DMA,
          pltpu.SemaphoreType.DMA,
          pltpu.SemaphoreType.DMA,
      ],
  )
  def lookup(aa_hbm, mask_hbm, table_hbm, out_hbm, idx_v, mask_v, rows_a,
             rows_b, sg_a, sg_b, ss_a, ss_b):
    wid = lax.axis_index("s") * 2 + lax.axis_index("c")
    base = wid * B_PER_W
    pltpu.sync_copy(aa_hbm.at[pl.ds(base, B_PER_W)], idx_v)
    pltpu.sync_copy(mask_hbm.at[pl.ds(base, B_PER_W)], mask_v)
    tab_base = wid * 8
    for j in range(B_PER_W // LANES):
      sl = pl.ds(j * LANES, LANES)
      a = idx_v[sl]
      m = mask_v[sl]
      idx_v[sl] = jnp.where(m == 0, jnp.int32(7), a) + tab_base
    rows = (rows_a, rows_b)
    sg = (sg_a, sg_b)
    ss = (ss_a, ss_b)
    g = [None] * N_CHUNKS
    s = [None] * N_CHUNKS
    for c in range(N_CHUNKS):
      b = c % 2
      if c >= 2:
        s[c - 2].wait()
      g[c] = pltpu.async_copy(
          table_hbm.at[idx_v.at[pl.ds(c * CHUNK, CHUNK)]], rows[b], sg[b]
      )
      if c >= 1:
        p = c - 1
        g[p].wait()
        s[p] = pltpu.async_copy(
            rows[p % 2], out_hbm.at[pl.ds(base + p * CHUNK, CHUNK)], ss[p % 2]
        )
    last = N_CHUNKS - 1
    g[last].wait()
    s[last] = pltpu.async_copy(
        rows[last % 2], out_hbm.at[pl.ds(base + last * CHUNK, CHUNK)],
        ss[last % 2]
    )
    s[last - 1].wait()
    s[last].wait()

  return lookup


_lookup = _make_lookup()


def kernel(aa, res_nb, chain_nb, pos_atoms, mask_atoms, fragment_type, emb_table):
  aa_flat = aa.reshape(B).astype(jnp.int32)
  mask_flat = mask_atoms[:, :, 0].reshape(B).astype(jnp.int32)
  table_ext = jnp.concatenate(
      [emb_table.astype(jnp.float32), jnp.zeros((1, D), jnp.float32)], axis=0
  )
  table_rep = jnp.tile(table_ext, (NUM_WORKERS, 1))
  out = _lookup(aa_flat, mask_flat, table_rep)
  return out.reshape(N, L, D)


# speedup vs baseline: 4.3957x; 1.2736x over previous
import functools

import jax
import jax.numpy as jnp
from jax import lax
from jax.experimental import pallas as pl
from jax.experimental.pallas import tpu as pltpu
from jax.experimental.pallas import tpu_sc as plsc

N, L, D = 32, 1024, 256
B = N * L
LANES = 16
NUM_WORKERS = 32
B_PER_W = B // NUM_WORKERS
CHUNK = 128
N_CHUNKS = B_PER_W // CHUNK
REPS = 4


def _make_lookup():
  mesh = plsc.VectorSubcoreMesh(core_axis_name="c", subcore_axis_name="s")

  @functools.partial(
      pl.kernel,
      mesh=mesh,
      out_type=jax.ShapeDtypeStruct((B, D), jnp.float32),
      scratch_types=[
          pltpu.VMEM((B_PER_W,), jnp.int32),
          pltpu.VMEM((B_PER_W,), jnp.int32),
          pltpu.VMEM((CHUNK, D), jnp.float32),
          pltpu.VMEM((CHUNK, D), jnp.float32),
          pltpu.SemaphoreType.DMA,
          pltpu.SemaphoreType.DMA,
          pltpu.SemaphoreType.DMA,
          pltpu.SemaphoreType.DMA,
      ],
  )
  def lookup(aa_hbm, mask_hbm, table_hbm, out_hbm, idx_v, mask_v, rows_a,
             rows_b, sg_a, sg_b, ss_a, ss_b):
    wid = lax.axis_index("s") * 2 + lax.axis_index("c")
    base = wid * B_PER_W
    pltpu.sync_copy(aa_hbm.at[pl.ds(base, B_PER_W)], idx_v)
    pltpu.sync_copy(mask_hbm.at[pl.ds(base, B_PER_W)], mask_v)
    tab_base = wid * (8 * REPS)
    for j in range(B_PER_W // LANES):
      sl = pl.ds(j * LANES, LANES)
      a = idx_v[sl]
      m = mask_v[sl]
      idx_v[sl] = jnp.where(m == 0, jnp.int32(7), a) + (
          tab_base + (j % REPS) * 8
      )
    rows = (rows_a, rows_b)
    sg = (sg_a, sg_b)
    ss = (ss_a, ss_b)
    g = [None] * N_CHUNKS
    s = [None] * N_CHUNKS
    for c in range(N_CHUNKS):
      b = c % 2
      if c >= 2:
        s[c - 2].wait()
      g[c] = pltpu.async_copy(
          table_hbm.at[idx_v.at[pl.ds(c * CHUNK, CHUNK)]], rows[b], sg[b]
      )
      if c >= 1:
        p = c - 1
        g[p].wait()
        s[p] = pltpu.async_copy(
            rows[p % 2], out_hbm.at[pl.ds(base + p * CHUNK, CHUNK)], ss[p % 2]
        )
    last = N_CHUNKS - 1
    g[last].wait()
    s[last] = pltpu.async_copy(
        rows[last % 2], out_hbm.at[pl.ds(base + last * CHUNK, CHUNK)],
        ss[last % 2]
    )
    s[last - 1].wait()
    s[last].wait()

  return lookup


_lookup = _make_lookup()


def kernel(aa, res_nb, chain_nb, pos_atoms, mask_atoms, fragment_type, emb_table):
  aa_flat = aa.reshape(B).astype(jnp.int32)
  mask_flat = mask_atoms[:, :, 0].reshape(B).astype(jnp.int32)
  table_ext = jnp.concatenate(
      [emb_table.astype(jnp.float32), jnp.zeros((1, D), jnp.float32)], axis=0
  )
  table_rep = jnp.tile(table_ext, (NUM_WORKERS * REPS, 1))
  out = _lookup(aa_flat, mask_flat, table_rep)
  return out.reshape(N, L, D)


# speedup vs baseline: 4.7928x; 1.0903x over previous
import functools

import jax
import jax.numpy as jnp
from jax import lax
from jax.experimental import pallas as pl
from jax.experimental.pallas import tpu as pltpu
from jax.experimental.pallas import tpu_sc as plsc

N, L, D = 32, 1024, 256
B = N * L
LANES = 16
NUM_WORKERS = 32
B_PER_W = B // NUM_WORKERS
CHUNK = 128
N_CHUNKS = B_PER_W // CHUNK
REPS = 8


def _make_lookup():
  mesh = plsc.VectorSubcoreMesh(core_axis_name="c", subcore_axis_name="s")

  @functools.partial(
      pl.kernel,
      mesh=mesh,
      out_type=jax.ShapeDtypeStruct((B, D), jnp.float32),
      scratch_types=[
          pltpu.VMEM((B_PER_W,), jnp.int32),
          pltpu.VMEM((B_PER_W,), jnp.int32),
          pltpu.VMEM((CHUNK, D), jnp.float32),
          pltpu.VMEM((CHUNK, D), jnp.float32),
          pltpu.SemaphoreType.DMA,
          pltpu.SemaphoreType.DMA,
          pltpu.SemaphoreType.DMA,
          pltpu.SemaphoreType.DMA,
      ],
  )
  def lookup(aa_hbm, mask_hbm, table_hbm, out_hbm, idx_v, mask_v, rows_a,
             rows_b, sg_a, sg_b, ss_a, ss_b):
    wid = lax.axis_index("s") * 2 + lax.axis_index("c")
    base = wid * B_PER_W
    pltpu.sync_copy(aa_hbm.at[pl.ds(base, B_PER_W)], idx_v)
    pltpu.sync_copy(mask_hbm.at[pl.ds(base, B_PER_W)], mask_v)
    tab_base = wid * (8 * REPS)
    for j in range(B_PER_W // LANES):
      sl = pl.ds(j * LANES, LANES)
      a = idx_v[sl]
      m = mask_v[sl]
      idx_v[sl] = jnp.where(m == 0, jnp.int32(7), a) + (
          tab_base + (j % REPS) * 8
      )
    rows = (rows_a, rows_b)
    sg = (sg_a, sg_b)
    ss = (ss_a, ss_b)
    g = [None] * N_CHUNKS
    s = [None] * N_CHUNKS
    for c in range(N_CHUNKS):
      b = c % 2
      if c >= 2:
        s[c - 2].wait()
      g[c] = pltpu.async_copy(
          table_hbm.at[idx_v.at[pl.ds(c * CHUNK, CHUNK)]], rows[b], sg[b]
      )
      if c >= 1:
        p = c - 1
        g[p].wait()
        s[p] = pltpu.async_copy(
            rows[p % 2], out_hbm.at[pl.ds(base + p * CHUNK, CHUNK)], ss[p % 2]
        )
    last = N_CHUNKS - 1
    g[last].wait()
    s[last] = pltpu.async_copy(
        rows[last % 2], out_hbm.at[pl.ds(base + last * CHUNK, CHUNK)],
        ss[last % 2]
    )
    s[last - 1].wait()
    s[last].wait()

  return lookup


_lookup = _make_lookup()


def kernel(aa, res_nb, chain_nb, pos_atoms, mask_atoms, fragment_type, emb_table):
  aa_flat = aa.reshape(B).astype(jnp.int32)
  mask_flat = mask_atoms[:, :, 0].reshape(B).astype(jnp.int32)
  table_ext = jnp.concatenate(
      [emb_table.astype(jnp.float32), jnp.zeros((1, D), jnp.float32)], axis=0
  )
  table_rep = jnp.tile(table_ext, (NUM_WORKERS * REPS, 1))
  out = _lookup(aa_flat, mask_flat, table_rep)
  return out.reshape(N, L, D)


# speedup vs baseline: 4.9729x; 1.0376x over previous
import functools

import jax
import jax.numpy as jnp
from jax import lax
from jax.experimental import pallas as pl
from jax.experimental.pallas import tpu as pltpu
from jax.experimental.pallas import tpu_sc as plsc

N, L, D = 32, 1024, 256
B = N * L
LANES = 16
NUM_WORKERS = 32
B_PER_W = B // NUM_WORKERS
CHUNK = 128
N_CHUNKS = B_PER_W // CHUNK
REPS = 16


def _make_lookup():
  mesh = plsc.VectorSubcoreMesh(core_axis_name="c", subcore_axis_name="s")

  @functools.partial(
      pl.kernel,
      mesh=mesh,
      out_type=jax.ShapeDtypeStruct((B, D), jnp.float32),
      scratch_types=[
          pltpu.VMEM((B_PER_W,), jnp.int32),
          pltpu.VMEM((B_PER_W,), jnp.int32),
          pltpu.VMEM((CHUNK, D), jnp.float32),
          pltpu.VMEM((CHUNK, D), jnp.float32),
          pltpu.SemaphoreType.DMA,
          pltpu.SemaphoreType.DMA,
          pltpu.SemaphoreType.DMA,
          pltpu.SemaphoreType.DMA,
      ],
  )
  def lookup(aa_hbm, mask_hbm, table_hbm, out_hbm, idx_v, mask_v, rows_a,
             rows_b, sg_a, sg_b, ss_a, ss_b):
    wid = lax.axis_index("s") * 2 + lax.axis_index("c")
    base = wid * B_PER_W
    pltpu.sync_copy(aa_hbm.at[pl.ds(base, B_PER_W)], idx_v)
    pltpu.sync_copy(mask_hbm.at[pl.ds(base, B_PER_W)], mask_v)
    tab_base = wid * (8 * REPS)
    for j in range(B_PER_W // LANES):
      sl = pl.ds(j * LANES, LANES)
      a = idx_v[sl]
      m = mask_v[sl]
      idx_v[sl] = jnp.where(m == 0, jnp.int32(7), a) + (
          tab_base + (j % REPS) * 8
      )
    rows = (rows_a, rows_b)
    sg = (sg_a, sg_b)
    ss = (ss_a, ss_b)
    g = [None] * N_CHUNKS
    s = [None] * N_CHUNKS
    for c in range(N_CHUNKS):
      b = c % 2
      if c >= 2:
        s[c - 2].wait()
      g[c] = pltpu.async_copy(
          table_hbm.at[idx_v.at[pl.ds(c * CHUNK, CHUNK)]], rows[b], sg[b]
      )
      if c >= 1:
        p = c - 1
        g[p].wait()
        s[p] = pltpu.async_copy(
            rows[p % 2], out_hbm.at[pl.ds(base + p * CHUNK, CHUNK)], ss[p % 2]
        )
    last = N_CHUNKS - 1
    g[last].wait()
    s[last] = pltpu.async_copy(
        rows[last % 2], out_hbm.at[pl.ds(base + last * CHUNK, CHUNK)],
        ss[last % 2]
    )
    s[last - 1].wait()
    s[last].wait()

  return lookup


_lookup = _make_lookup()


def kernel(aa, res_nb, chain_nb, pos_atoms, mask_atoms, fragment_type, emb_table):
  aa_flat = aa.reshape(B).astype(jnp.int32)
  mask_flat = mask_atoms[:, :, 0].reshape(B).astype(jnp.int32)
  table_ext = jnp.concatenate(
      [emb_table.astype(jnp.float32), jnp.zeros((1, D), jnp.float32)], axis=0
  )
  table_rep = jnp.tile(table_ext, (NUM_WORKERS * REPS, 1))
  out = _lookup(aa_flat, mask_flat, table_rep)
  return out.reshape(N, L, D)
